# baseline (speedup 1.0000x reference)
import jax
import jax.numpy as jnp
from jax.experimental import pallas as pl

_BLK = 1024


def _encode_block(p_ref, wk_ref, bk_ref, wv_ref, bv_ref, k_ref, v_ref):
    p = p_ref[...]
    k_ref[...] = (
        jnp.dot(p, wk_ref[...], preferred_element_type=jnp.float32) + bk_ref[...]
    )
    v_ref[...] = (
        jnp.dot(p, wv_ref[...], preferred_element_type=jnp.float32) + bv_ref[...]
    )


def kernel(x, labels, prototype_vectors, Wk, bk, Wv, bv):
    n, d = prototype_vectors.shape
    a = Wk.shape[0]
    wk_t = Wk.T
    wv_t = Wv.T
    bk2 = bk.reshape(1, a)
    bv2 = bv.reshape(1, a)
    k, v = pl.pallas_call(
        _encode_block,
        grid=(n // _BLK,),
        in_specs=[
            pl.BlockSpec((_BLK, d), lambda i: (i, 0)),
            pl.BlockSpec((d, a), lambda i: (0, 0)),
            pl.BlockSpec((1, a), lambda i: (0, 0)),
            pl.BlockSpec((d, a), lambda i: (0, 0)),
            pl.BlockSpec((1, a), lambda i: (0, 0)),
        ],
        out_specs=[
            pl.BlockSpec((_BLK, a), lambda i: (i, 0)),
            pl.BlockSpec((_BLK, a), lambda i: (i, 0)),
        ],
        out_shape=[
            jax.ShapeDtypeStruct((n, a), jnp.float32),
            jax.ShapeDtypeStruct((n, a), jnp.float32),
        ],
    )(prototype_vectors, wk_t, bk2, wv_t, bv2)
    return (k, v)


# speedup vs baseline: 1.2958x; 1.2958x over previous
import jax
import jax.numpy as jnp
from jax.experimental import pallas as pl


def _encode_block(p_ref, wk_ref, bk_ref, wv_ref, bv_ref, k_ref, v_ref):
    p = p_ref[...]
    k_ref[...] = (
        jnp.dot(p, wk_ref[...].T, preferred_element_type=jnp.float32) + bk_ref[...]
    )
    v_ref[...] = (
        jnp.dot(p, wv_ref[...].T, preferred_element_type=jnp.float32) + bv_ref[...]
    )


def kernel(x, labels, prototype_vectors, Wk, bk, Wv, bv):
    n, d = prototype_vectors.shape
    a = Wk.shape[0]
    bk2 = bk.reshape(1, a)
    bv2 = bv.reshape(1, a)
    k, v = pl.pallas_call(
        _encode_block,
        out_shape=[
            jax.ShapeDtypeStruct((n, a), jnp.float32),
            jax.ShapeDtypeStruct((n, a), jnp.float32),
        ],
    )(prototype_vectors, Wk, bk2, Wv, bv2)
    return (k, v)
